# Initial kernel scaffold; baseline (speedup 1.0000x reference)
import jax
import jax.numpy as jnp
from jax.experimental import pallas as pl


def kernel(g, x, W_est, b_est, W_gnn, b_gnn, W_cls, b_cls):
    raise NotImplementedError("write your pallas kernel here")


# speedup vs baseline: 97.6193x; 97.6193x over previous
import functools

import jax
import jax.numpy as jnp
from jax import lax
from jax.experimental import pallas as pl
from jax.experimental.pallas import tpu as pltpu
from jax.experimental.pallas import tpu_sc as plsc

N = 10000
NFEAT = 128
E = 320000

NC = 2
NT = 16
NW = NC * NT
LANES = 16

NPAD = 10240
ECHUNK = E // NW
COLS = NPAD // NW
EPASS = 5
EPIECE = ECHUNK // EPASS

_SC_PARAMS = pltpu.CompilerParams(use_tc_tiling_on_sc=False,
                                  needs_layout_passes=False)


def _deg_body(dst_hbm, part_hbm, dst_v, acc_v):
    cid = lax.axis_index("c")
    sid = lax.axis_index("s")
    wid = cid * NT + sid
    pltpu.sync_copy(dst_hbm.at[pl.ds(wid * ECHUNK, ECHUNK)], dst_v)

    zero16 = jnp.zeros((LANES,), jnp.float32)

    def zb(i, _):
        acc_v[pl.ds(i * LANES, LANES)] = zero16
        return 0
    lax.fori_loop(0, NPAD // LANES, zb, 0)

    one16 = jnp.ones((LANES,), jnp.float32)

    def cb(i, _):
        d = dst_v[pl.ds(i * LANES, LANES)]
        plsc.addupdate_scatter(acc_v, [d], one16)
        return 0
    lax.fori_loop(0, ECHUNK // LANES, cb, 0)

    pltpu.sync_copy(acc_v, part_hbm.at[wid])


def _tca_body(part_ref, x_ref, we_ref, wg_ref, wc_ref, be_ref, bg_ref,
              bc_ref, xw_ref, dinv_ref, bias_ref):
    deg = jnp.sum(part_ref[...], axis=0, keepdims=True) + 1.0
    dinv_ref[...] = lax.rsqrt(deg)
    wy = jnp.dot(wg_ref[...], wc_ref[...], preferred_element_type=jnp.float32,
                 precision=lax.Precision.HIGHEST)
    wcat = jnp.concatenate([we_ref[...], wy], axis=1)
    xw_ref[...] = jnp.dot(x_ref[...], wcat,
                          preferred_element_type=jnp.float32,
                          precision=lax.Precision.HIGHEST)
    by = jnp.dot(bg_ref[...], wc_ref[...],
                 preferred_element_type=jnp.float32,
                 precision=lax.Precision.HIGHEST) + bc_ref[...]
    bias_ref[...] = jnp.concatenate(
        [jnp.broadcast_to(be_ref[...], (1, LANES)),
         jnp.broadcast_to(by, (1, LANES))], axis=0)


def _msg_body(src_hbm, dst_hbm, xw_hbm, dinv_hbm, part_hbm,
              src_v, dst_v, xw_v, dinv_v, acc0_v, acc1_v):
    cid = lax.axis_index("c")
    sid = lax.axis_index("s")
    wid = cid * NT + sid
    pltpu.sync_copy(xw_hbm, xw_v)
    pltpu.sync_copy(dinv_hbm, dinv_v)

    zero16 = jnp.zeros((LANES,), jnp.float32)

    def zb(i, _):
        acc0_v[pl.ds(i * LANES, LANES)] = zero16
        acc1_v[pl.ds(i * LANES, LANES)] = zero16
        return 0
    lax.fori_loop(0, NPAD // LANES, zb, 0)

    c0 = jnp.zeros((LANES,), jnp.int32)
    c1 = jnp.full((LANES,), 1, jnp.int32)

    def eb(i, _):
        s = src_v[pl.ds(i * LANES, LANES)]
        d = dst_v[pl.ds(i * LANES, LANES)]
        xs0 = plsc.load_gather(xw_v, [s, c0])
        xs1 = plsc.load_gather(xw_v, [s, c1])
        ns = plsc.load_gather(dinv_v, [s])
        nd = plsc.load_gather(dinv_v, [d])
        f = ns * nd
        plsc.addupdate_scatter(acc0_v, [d], xs0 * f)
        plsc.addupdate_scatter(acc1_v, [d], xs1 * f)
        return 0

    for p in range(EPASS):
        off = wid * ECHUNK + p * EPIECE
        pltpu.sync_copy(src_hbm.at[pl.ds(off, EPIECE)], src_v)
        pltpu.sync_copy(dst_hbm.at[pl.ds(off, EPIECE)], dst_v)
        lax.fori_loop(0, EPIECE // LANES, eb, 0)

    iota16 = lax.iota(jnp.int32, LANES)

    def sb(j, _):
        idx = wid * COLS + j * LANES + iota16
        xs0 = plsc.load_gather(xw_v, [idx, c0])
        xs1 = plsc.load_gather(xw_v, [idx, c1])
        nn = plsc.load_gather(dinv_v, [idx])
        n2 = nn * nn
        plsc.addupdate_scatter(acc0_v, [idx], xs0 * n2)
        plsc.addupdate_scatter(acc1_v, [idx], xs1 * n2)
        return 0
    lax.fori_loop(0, COLS // LANES, sb, 0)

    pltpu.sync_copy(acc0_v, part_hbm.at[wid, 0])
    pltpu.sync_copy(acc1_v, part_hbm.at[wid, 1])


def _tcb_body(part_ref, bias_ref, out_ref):
    r = jnp.sum(part_ref[...], axis=0)
    out_ref[...] = r + bias_ref[...][:, 0:1]


@functools.cache
def _build_calls():
    mesh = plsc.VectorSubcoreMesh(core_axis_name="c", subcore_axis_name="s",
                                  num_cores=NC, num_subcores=NT)
    deg_call = pl.kernel(
        _deg_body,
        out_type=jax.ShapeDtypeStruct((NW, NPAD), jnp.float32),
        mesh=mesh,
        scratch_types=[
            pltpu.VMEM((ECHUNK,), jnp.int32),
            pltpu.VMEM((NPAD,), jnp.float32),
        ],
        compiler_params=_SC_PARAMS,
    )
    tca_call = pl.pallas_call(
        _tca_body,
        out_shape=[
            jax.ShapeDtypeStruct((NPAD, 2), jnp.float32),
            jax.ShapeDtypeStruct((1, NPAD), jnp.float32),
            jax.ShapeDtypeStruct((2, LANES), jnp.float32),
        ],
    )
    msg_call = pl.kernel(
        _msg_body,
        out_type=jax.ShapeDtypeStruct((NW, 2, NPAD), jnp.float32),
        mesh=mesh,
        scratch_types=[
            pltpu.VMEM((EPIECE,), jnp.int32),
            pltpu.VMEM((EPIECE,), jnp.int32),
            pltpu.VMEM((NPAD, 2), jnp.float32),
            pltpu.VMEM((NPAD,), jnp.float32),
            pltpu.VMEM((NPAD,), jnp.float32),
            pltpu.VMEM((NPAD,), jnp.float32),
        ],
        compiler_params=_SC_PARAMS,
    )
    tcb_call = pl.pallas_call(
        _tcb_body,
        out_shape=jax.ShapeDtypeStruct((2, NPAD), jnp.float32),
    )
    return deg_call, tca_call, msg_call, tcb_call


def kernel(g, x, W_est, b_est, W_gnn, b_gnn, W_cls, b_cls):
    deg_call, tca_call, msg_call, tcb_call = _build_calls()
    src = g[0]
    dst = g[1]
    x_pad = jnp.concatenate(
        [x, jnp.zeros((NPAD - N, NFEAT), jnp.float32)], axis=0)
    deg_parts = deg_call(dst)
    xw, dinv2, biasrow = tca_call(deg_parts, x_pad, W_est, W_gnn, W_cls,
                                  b_est.reshape(1, 1), b_gnn.reshape(1, -1),
                                  b_cls.reshape(1, 1))
    msg_parts = msg_call(src, dst, xw, dinv2.reshape(NPAD))
    out2 = tcb_call(msg_parts, biasrow)
    s = out2[0, :N].reshape(N, 1)
    y = out2[1, :N].reshape(N, 1)
    return (y, s)


# speedup vs baseline: 112.3122x; 1.1505x over previous
import functools

import jax
import jax.numpy as jnp
from jax import lax
from jax.experimental import pallas as pl
from jax.experimental.pallas import tpu as pltpu
from jax.experimental.pallas import tpu_sc as plsc

N = 10000
NFEAT = 128
E = 320000

NC = 2
NT = 16
NW = NC * NT
LANES = 16

NPAD = 10240
ECHUNK = E // NW
COLS = NPAD // NW
EPASS = 5
EPIECE = ECHUNK // EPASS

_SC_PARAMS = pltpu.CompilerParams(use_tc_tiling_on_sc=False,
                                  needs_layout_passes=False)


def _deg_body(g_hbm, part_hbm, dst_v, acc_v):
    cid = lax.axis_index("c")
    sid = lax.axis_index("s")
    wid = cid * NT + sid
    pltpu.sync_copy(g_hbm.at[1, pl.ds(wid * ECHUNK, ECHUNK)], dst_v)

    zero16 = jnp.zeros((LANES,), jnp.float32)

    def zb(i, _):
        acc_v[pl.ds(i * LANES, LANES)] = zero16
        return 0
    lax.fori_loop(0, NPAD // LANES, zb, 0, unroll=8)

    one16 = jnp.ones((LANES,), jnp.float32)

    def cb(i, _):
        d = dst_v[pl.ds(i * LANES, LANES)]
        plsc.addupdate_scatter(acc_v, [d], one16)
        return 0
    lax.fori_loop(0, ECHUNK // LANES, cb, 0, unroll=8)

    pltpu.sync_copy(acc_v, part_hbm.at[wid])


def _tca_body(part_ref, x_ref, we_ref, wg_ref, wc_ref, be_ref, bg_ref,
              bc_ref, xw_ref, dinv_ref, bias_ref):
    deg = jnp.sum(part_ref[...], axis=0, keepdims=True) + 1.0
    dinv_ref[...] = lax.rsqrt(deg)
    wy = jnp.dot(wg_ref[...], wc_ref[...], preferred_element_type=jnp.float32,
                 precision=lax.Precision.HIGHEST)
    wcat = jnp.concatenate([we_ref[...], wy], axis=1)
    xw_ref[0:N, :] = jnp.dot(x_ref[...], wcat,
                             preferred_element_type=jnp.float32,
                             precision=lax.Precision.HIGHEST)
    xw_ref[N:NPAD, :] = jnp.zeros((NPAD - N, 2), jnp.float32)
    by = jnp.dot(bg_ref[...], wc_ref[...],
                 preferred_element_type=jnp.float32,
                 precision=lax.Precision.HIGHEST) + bc_ref[...]
    bias_ref[...] = jnp.concatenate(
        [jnp.broadcast_to(be_ref[...], (1, LANES)),
         jnp.broadcast_to(by, (1, LANES))], axis=0)


def _msg_body(g_hbm, xw_hbm, dinv_hbm, part_hbm,
              src_v, dst_v, xw_v, dinv_v, acc0_v, acc1_v):
    cid = lax.axis_index("c")
    sid = lax.axis_index("s")
    wid = cid * NT + sid
    pltpu.sync_copy(xw_hbm, xw_v)
    pltpu.sync_copy(dinv_hbm, dinv_v)

    zero16 = jnp.zeros((LANES,), jnp.float32)

    def zb(i, _):
        acc0_v[pl.ds(i * LANES, LANES)] = zero16
        acc1_v[pl.ds(i * LANES, LANES)] = zero16
        return 0
    lax.fori_loop(0, NPAD // LANES, zb, 0, unroll=8)

    c0 = jnp.zeros((LANES,), jnp.int32)
    c1 = jnp.full((LANES,), 1, jnp.int32)

    def eb(i, _):
        s = src_v[pl.ds(i * LANES, LANES)]
        d = dst_v[pl.ds(i * LANES, LANES)]
        xs0 = plsc.load_gather(xw_v, [s, c0])
        xs1 = plsc.load_gather(xw_v, [s, c1])
        ns = plsc.load_gather(dinv_v, [s])
        nd = plsc.load_gather(dinv_v, [d])
        f = ns * nd
        plsc.addupdate_scatter(acc0_v, [d], xs0 * f)
        plsc.addupdate_scatter(acc1_v, [d], xs1 * f)
        return 0

    for p in range(EPASS):
        off = wid * ECHUNK + p * EPIECE
        pltpu.sync_copy(g_hbm.at[0, pl.ds(off, EPIECE)], src_v)
        pltpu.sync_copy(g_hbm.at[1, pl.ds(off, EPIECE)], dst_v)
        lax.fori_loop(0, EPIECE // LANES, eb, 0, unroll=8)

    iota16 = lax.iota(jnp.int32, LANES)

    def sb(j, _):
        idx = wid * COLS + j * LANES + iota16
        xs0 = plsc.load_gather(xw_v, [idx, c0])
        xs1 = plsc.load_gather(xw_v, [idx, c1])
        nn = plsc.load_gather(dinv_v, [idx])
        n2 = nn * nn
        plsc.addupdate_scatter(acc0_v, [idx], xs0 * n2)
        plsc.addupdate_scatter(acc1_v, [idx], xs1 * n2)
        return 0
    lax.fori_loop(0, COLS // LANES, sb, 0, unroll=4)

    pltpu.sync_copy(acc0_v, part_hbm.at[wid, 0])
    pltpu.sync_copy(acc1_v, part_hbm.at[wid, 1])


def _tcb_body(part_ref, bias_ref, out_ref):
    r = jnp.sum(part_ref[...], axis=0)
    out_ref[...] = r + bias_ref[...][:, 0:1]


@functools.cache
def _build_calls():
    mesh = plsc.VectorSubcoreMesh(core_axis_name="c", subcore_axis_name="s",
                                  num_cores=NC, num_subcores=NT)
    deg_call = pl.kernel(
        _deg_body,
        out_type=jax.ShapeDtypeStruct((NW, NPAD), jnp.float32),
        mesh=mesh,
        scratch_types=[
            pltpu.VMEM((ECHUNK,), jnp.int32),
            pltpu.VMEM((NPAD,), jnp.float32),
        ],
        compiler_params=_SC_PARAMS,
    )
    tca_call = pl.pallas_call(
        _tca_body,
        out_shape=[
            jax.ShapeDtypeStruct((NPAD, 2), jnp.float32),
            jax.ShapeDtypeStruct((1, NPAD), jnp.float32),
            jax.ShapeDtypeStruct((2, LANES), jnp.float32),
        ],
    )
    msg_call = pl.kernel(
        _msg_body,
        out_type=jax.ShapeDtypeStruct((NW, 2, NPAD), jnp.float32),
        mesh=mesh,
        scratch_types=[
            pltpu.VMEM((EPIECE,), jnp.int32),
            pltpu.VMEM((EPIECE,), jnp.int32),
            pltpu.VMEM((NPAD, 2), jnp.float32),
            pltpu.VMEM((NPAD,), jnp.float32),
            pltpu.VMEM((NPAD,), jnp.float32),
            pltpu.VMEM((NPAD,), jnp.float32),
        ],
        compiler_params=_SC_PARAMS,
    )
    tcb_call = pl.pallas_call(
        _tcb_body,
        out_shape=jax.ShapeDtypeStruct((2, NPAD), jnp.float32),
    )
    return deg_call, tca_call, msg_call, tcb_call


def kernel(g, x, W_est, b_est, W_gnn, b_gnn, W_cls, b_cls):
    deg_call, tca_call, msg_call, tcb_call = _build_calls()
    deg_parts = deg_call(g)
    xw, dinv2, biasrow = tca_call(deg_parts, x, W_est, W_gnn, W_cls,
                                  b_est.reshape(1, 1), b_gnn.reshape(1, -1),
                                  b_cls.reshape(1, 1))
    msg_parts = msg_call(g, xw, dinv2.reshape(NPAD))
    out2 = tcb_call(msg_parts, biasrow)
    s = out2[0, :N].reshape(N, 1)
    y = out2[1, :N].reshape(N, 1)
    return (y, s)


# speedup vs baseline: 121.8393x; 1.0848x over previous
import functools

import jax
import jax.numpy as jnp
from jax import lax
from jax.experimental import pallas as pl
from jax.experimental.pallas import tpu as pltpu
from jax.experimental.pallas import tpu_sc as plsc

N = 10000
NFEAT = 128
E = 320000

NC = 2
NT = 16
NW = NC * NT
LANES = 16

NPAD = 10240
ECHUNK = E // NW
COLS = NPAD // NW
EPASS = 5
EPIECE = ECHUNK // EPASS

_SC_PARAMS = pltpu.CompilerParams(use_tc_tiling_on_sc=False,
                                  needs_layout_passes=False)


def _deg_body(g_hbm, part_hbm, dst_v, acc_v):
    cid = lax.axis_index("c")
    sid = lax.axis_index("s")
    wid = cid * NT + sid
    pltpu.sync_copy(g_hbm.at[1, pl.ds(wid * ECHUNK, ECHUNK)], dst_v)

    zero16 = jnp.zeros((LANES,), jnp.float32)

    def zb(i, _):
        acc_v[pl.ds(i * LANES, LANES)] = zero16
        return 0
    lax.fori_loop(0, NPAD // LANES, zb, 0, unroll=8)

    one16 = jnp.ones((LANES,), jnp.float32)

    @plsc.parallel_loop(0, ECHUNK // LANES, unroll=8)
    def cb(i):
        d = dst_v[pl.ds(i * LANES, LANES)]
        plsc.addupdate_scatter(acc_v, [d], one16)

    pltpu.sync_copy(acc_v, part_hbm.at[wid])


def _tca_body(part_ref, x_ref, we_ref, wg_ref, wc_ref, be_ref, bg_ref,
              bc_ref, xw_ref, dinv_ref, bias_ref):
    deg = jnp.sum(part_ref[...], axis=0, keepdims=True) + 1.0
    dinv_ref[...] = lax.rsqrt(deg)
    wy = jnp.dot(wg_ref[...], wc_ref[...], preferred_element_type=jnp.float32,
                 precision=lax.Precision.HIGHEST)
    wcat = jnp.concatenate([we_ref[...], wy], axis=1)
    xw_ref[0:N, :] = jnp.dot(x_ref[...], wcat,
                             preferred_element_type=jnp.float32,
                             precision=lax.Precision.HIGHEST)
    xw_ref[N:NPAD, :] = jnp.zeros((NPAD - N, 2), jnp.float32)
    by = jnp.dot(bg_ref[...], wc_ref[...],
                 preferred_element_type=jnp.float32,
                 precision=lax.Precision.HIGHEST) + bc_ref[...]
    bias_ref[...] = jnp.concatenate(
        [jnp.broadcast_to(be_ref[...], (1, LANES)),
         jnp.broadcast_to(by, (1, LANES))], axis=0)


def _msg_body(g_hbm, xw_hbm, dinv_hbm, part_hbm,
              src_v, dst_v, xw_v, dinv_v, acc0_v, acc1_v):
    cid = lax.axis_index("c")
    sid = lax.axis_index("s")
    wid = cid * NT + sid
    pltpu.sync_copy(xw_hbm, xw_v)
    pltpu.sync_copy(dinv_hbm, dinv_v)

    zero16 = jnp.zeros((LANES,), jnp.float32)

    def zb(i, _):
        acc0_v[pl.ds(i * LANES, LANES)] = zero16
        acc1_v[pl.ds(i * LANES, LANES)] = zero16
        return 0
    lax.fori_loop(0, NPAD // LANES, zb, 0, unroll=8)

    c0 = jnp.zeros((LANES,), jnp.int32)
    c1 = jnp.full((LANES,), 1, jnp.int32)

    def eb(i):
        s = src_v[pl.ds(i * LANES, LANES)]
        d = dst_v[pl.ds(i * LANES, LANES)]
        xs0 = plsc.load_gather(xw_v, [s, c0])
        xs1 = plsc.load_gather(xw_v, [s, c1])
        ns = plsc.load_gather(dinv_v, [s])
        nd = plsc.load_gather(dinv_v, [d])
        f = ns * nd
        plsc.addupdate_scatter(acc0_v, [d], xs0 * f)
        plsc.addupdate_scatter(acc1_v, [d], xs1 * f)

    for p in range(EPASS):
        off = wid * ECHUNK + p * EPIECE
        pltpu.sync_copy(g_hbm.at[0, pl.ds(off, EPIECE)], src_v)
        pltpu.sync_copy(g_hbm.at[1, pl.ds(off, EPIECE)], dst_v)
        plsc.parallel_loop(0, EPIECE // LANES, unroll=8)(eb)

    iota16 = lax.iota(jnp.int32, LANES)

    def sb(j, _):
        idx = wid * COLS + j * LANES + iota16
        xs0 = plsc.load_gather(xw_v, [idx, c0])
        xs1 = plsc.load_gather(xw_v, [idx, c1])
        nn = plsc.load_gather(dinv_v, [idx])
        n2 = nn * nn
        plsc.addupdate_scatter(acc0_v, [idx], xs0 * n2)
        plsc.addupdate_scatter(acc1_v, [idx], xs1 * n2)
        return 0
    lax.fori_loop(0, COLS // LANES, sb, 0, unroll=4)

    pltpu.sync_copy(acc0_v, part_hbm.at[wid, 0])
    pltpu.sync_copy(acc1_v, part_hbm.at[wid, 1])


def _tcb_body(part_ref, bias_ref, out_ref):
    r = jnp.sum(part_ref[...], axis=0)
    out_ref[...] = r + bias_ref[...][:, 0:1]


@functools.cache
def _build_calls():
    mesh = plsc.VectorSubcoreMesh(core_axis_name="c", subcore_axis_name="s",
                                  num_cores=NC, num_subcores=NT)
    deg_call = pl.kernel(
        _deg_body,
        out_type=jax.ShapeDtypeStruct((NW, NPAD), jnp.float32),
        mesh=mesh,
        scratch_types=[
            pltpu.VMEM((ECHUNK,), jnp.int32),
            pltpu.VMEM((NPAD,), jnp.float32),
        ],
        compiler_params=_SC_PARAMS,
    )
    tca_call = pl.pallas_call(
        _tca_body,
        out_shape=[
            jax.ShapeDtypeStruct((NPAD, 2), jnp.float32),
            jax.ShapeDtypeStruct((1, NPAD), jnp.float32),
            jax.ShapeDtypeStruct((2, LANES), jnp.float32),
        ],
    )
    msg_call = pl.kernel(
        _msg_body,
        out_type=jax.ShapeDtypeStruct((NW, 2, NPAD), jnp.float32),
        mesh=mesh,
        scratch_types=[
            pltpu.VMEM((EPIECE,), jnp.int32),
            pltpu.VMEM((EPIECE,), jnp.int32),
            pltpu.VMEM((NPAD, 2), jnp.float32),
            pltpu.VMEM((NPAD,), jnp.float32),
            pltpu.VMEM((NPAD,), jnp.float32),
            pltpu.VMEM((NPAD,), jnp.float32),
        ],
        compiler_params=_SC_PARAMS,
    )
    tcb_call = pl.pallas_call(
        _tcb_body,
        out_shape=jax.ShapeDtypeStruct((2, NPAD), jnp.float32),
    )
    return deg_call, tca_call, msg_call, tcb_call


def kernel(g, x, W_est, b_est, W_gnn, b_gnn, W_cls, b_cls):
    deg_call, tca_call, msg_call, tcb_call = _build_calls()
    deg_parts = deg_call(g)
    xw, dinv2, biasrow = tca_call(deg_parts, x, W_est, W_gnn, W_cls,
                                  b_est.reshape(1, 1), b_gnn.reshape(1, -1),
                                  b_cls.reshape(1, 1))
    msg_parts = msg_call(g, xw, dinv2.reshape(NPAD))
    out2 = tcb_call(msg_parts, biasrow)
    s = out2[0, :N].reshape(N, 1)
    y = out2[1, :N].reshape(N, 1)
    return (y, s)


# speedup vs baseline: 135.2227x; 1.1098x over previous
import functools

import jax
import jax.numpy as jnp
from jax import lax
from jax.experimental import pallas as pl
from jax.experimental.pallas import tpu as pltpu
from jax.experimental.pallas import tpu_sc as plsc

N = 10000
NFEAT = 128
E = 320000

NC = 2
NT = 16
NW = NC * NT
LANES = 16

NPAD = 10240
ECHUNK16 = E // NT
ECHUNK32 = E // NW
EPIECE = 2000
COLS16 = NPAD // NT
COLS32 = NPAD // NW

_SC_PARAMS = pltpu.CompilerParams(use_tc_tiling_on_sc=False,
                                  needs_layout_passes=False)


def _rsqrt_newton(t):
    h = t * 0.5
    bi = plsc.bitcast(t, jnp.int32)
    bi = 0x5F3759DF - lax.shift_right_logical(bi, 1)
    r = plsc.bitcast(bi, jnp.float32)
    r = r * (1.5 - h * r * r)
    r = r * (1.5 - h * r * r)
    r = r * (1.5 - h * r * r)
    return r


def _deg_body(g_hbm, part_hbm, dinv_hbm, dst_v, acc_v, red_v, res_v):
    cid = lax.axis_index("c")
    sid = lax.axis_index("s")
    wid = cid * NT + sid

    zero16 = jnp.zeros((LANES,), jnp.float32)

    def zb(i, _):
        acc_v[pl.ds(i * LANES, LANES)] = zero16
        return 0
    lax.fori_loop(0, NPAD // LANES, zb, 0, unroll=8)

    one16 = jnp.ones((LANES,), jnp.float32)

    def cb(i):
        d = dst_v[pl.ds(i * LANES, LANES)]
        plsc.addupdate_scatter(acc_v, [d], one16)

    for p in range(ECHUNK16 // EPIECE):
        pltpu.sync_copy(
            g_hbm.at[1, pl.ds(sid * ECHUNK16 + p * EPIECE, EPIECE)], dst_v)
        plsc.parallel_loop(0, EPIECE // LANES, unroll=8)(cb)

    pltpu.sync_copy(acc_v, part_hbm.at[cid, sid])
    plsc.subcore_barrier()
    pltpu.sync_copy(part_hbm.at[cid, :, pl.ds(wid * COLS32, COLS32)], red_v)

    def rb(j, _):
        t = jnp.full((LANES,), 1.0, jnp.float32)
        for tt in range(NT):
            t = t + red_v[tt, pl.ds(j * LANES, LANES)]
        res_v[pl.ds(j * LANES, LANES)] = _rsqrt_newton(t)
        return 0
    lax.fori_loop(0, COLS32 // LANES, rb, 0)

    pltpu.sync_copy(res_v, dinv_hbm.at[pl.ds(wid * COLS32, COLS32)])


def _tca_body(x_ref, we_ref, wg_ref, wc_ref, be_ref, bg_ref, bc_ref,
              xw_ref, bias_ref):
    wy = jnp.dot(wg_ref[...], wc_ref[...], preferred_element_type=jnp.float32,
                 precision=lax.Precision.HIGHEST)
    wcat = jnp.concatenate([we_ref[...], wy], axis=1)
    xw_ref[0:N, :] = jnp.dot(x_ref[...], wcat,
                             preferred_element_type=jnp.float32,
                             precision=lax.Precision.HIGHEST)
    xw_ref[N:NPAD, :] = jnp.zeros((NPAD - N, 2), jnp.float32)
    by = jnp.dot(bg_ref[...], wc_ref[...],
                 preferred_element_type=jnp.float32,
                 precision=lax.Precision.HIGHEST) + bc_ref[...]
    bias_ref[...] = jnp.concatenate(
        [jnp.broadcast_to(be_ref[...], (1, LANES)),
         jnp.broadcast_to(by, (1, LANES))], axis=0)


def _msg_body(g_hbm, xw_hbm, dinv_hbm, part_hbm, psum_hbm,
              src_v, dst_v, xw_v, dinv_v, acc0_v, acc1_v, red_v, res_v):
    cid = lax.axis_index("c")
    sid = lax.axis_index("s")
    wid = cid * NT + sid
    pltpu.sync_copy(xw_hbm, xw_v)
    pltpu.sync_copy(dinv_hbm, dinv_v)

    zero16 = jnp.zeros((LANES,), jnp.float32)

    def zb(i, _):
        acc0_v[pl.ds(i * LANES, LANES)] = zero16
        acc1_v[pl.ds(i * LANES, LANES)] = zero16
        return 0
    lax.fori_loop(0, NPAD // LANES, zb, 0, unroll=8)

    c0 = jnp.zeros((LANES,), jnp.int32)
    c1 = jnp.full((LANES,), 1, jnp.int32)

    def eb(i):
        s = src_v[pl.ds(i * LANES, LANES)]
        d = dst_v[pl.ds(i * LANES, LANES)]
        xs0 = plsc.load_gather(xw_v, [s, c0])
        xs1 = plsc.load_gather(xw_v, [s, c1])
        ns = plsc.load_gather(dinv_v, [s])
        nd = plsc.load_gather(dinv_v, [d])
        f = ns * nd
        plsc.addupdate_scatter(acc0_v, [d], xs0 * f)
        plsc.addupdate_scatter(acc1_v, [d], xs1 * f)

    for p in range(ECHUNK32 // EPIECE):
        off = wid * ECHUNK32 + p * EPIECE
        pltpu.sync_copy(g_hbm.at[0, pl.ds(off, EPIECE)], src_v)
        pltpu.sync_copy(g_hbm.at[1, pl.ds(off, EPIECE)], dst_v)
        plsc.parallel_loop(0, EPIECE // LANES, unroll=8)(eb)

    iota16 = lax.iota(jnp.int32, LANES)

    def sb(j, _):
        idx = wid * COLS32 + j * LANES + iota16
        xs0 = plsc.load_gather(xw_v, [idx, c0])
        xs1 = plsc.load_gather(xw_v, [idx, c1])
        nn = plsc.load_gather(dinv_v, [idx])
        n2 = nn * nn
        plsc.addupdate_scatter(acc0_v, [idx], xs0 * n2)
        plsc.addupdate_scatter(acc1_v, [idx], xs1 * n2)
        return 0
    lax.fori_loop(0, COLS32 // LANES, sb, 0, unroll=4)

    pltpu.sync_copy(acc0_v, part_hbm.at[wid, 0])
    pltpu.sync_copy(acc1_v, part_hbm.at[wid, 1])
    plsc.subcore_barrier()
    for c in range(2):
        pltpu.sync_copy(
            part_hbm.at[pl.ds(cid * NT, NT), c, pl.ds(sid * COLS16, COLS16)],
            red_v)

        def rb(j, _):
            t = jnp.zeros((LANES,), jnp.float32)
            for tt in range(NT):
                t = t + red_v[tt, pl.ds(j * LANES, LANES)]
            res_v[pl.ds(j * LANES, LANES)] = t
            return 0
        lax.fori_loop(0, COLS16 // LANES, rb, 0)
        pltpu.sync_copy(res_v,
                        psum_hbm.at[cid, c, pl.ds(sid * COLS16, COLS16)])


def _tcb_body(psum_ref, bias_ref, out_ref):
    p = psum_ref[...]
    out_ref[...] = p[0] + p[1] + bias_ref[...][:, 0:1]


@functools.cache
def _build_calls():
    mesh = plsc.VectorSubcoreMesh(core_axis_name="c", subcore_axis_name="s",
                                  num_cores=NC, num_subcores=NT)
    deg_call = pl.kernel(
        _deg_body,
        out_type=[
            jax.ShapeDtypeStruct((NC, NT, NPAD), jnp.float32),
            jax.ShapeDtypeStruct((NPAD,), jnp.float32),
        ],
        mesh=mesh,
        scratch_types=[
            pltpu.VMEM((EPIECE,), jnp.int32),
            pltpu.VMEM((NPAD,), jnp.float32),
            pltpu.VMEM((NT, COLS32), jnp.float32),
            pltpu.VMEM((COLS32,), jnp.float32),
        ],
        compiler_params=_SC_PARAMS,
    )
    tca_call = pl.pallas_call(
        _tca_body,
        out_shape=[
            jax.ShapeDtypeStruct((NPAD, 2), jnp.float32),
            jax.ShapeDtypeStruct((2, LANES), jnp.float32),
        ],
    )
    msg_call = pl.kernel(
        _msg_body,
        out_type=[
            jax.ShapeDtypeStruct((NW, 2, NPAD), jnp.float32),
            jax.ShapeDtypeStruct((NC, 2, NPAD), jnp.float32),
        ],
        mesh=mesh,
        scratch_types=[
            pltpu.VMEM((EPIECE,), jnp.int32),
            pltpu.VMEM((EPIECE,), jnp.int32),
            pltpu.VMEM((NPAD, 2), jnp.float32),
            pltpu.VMEM((NPAD,), jnp.float32),
            pltpu.VMEM((NPAD,), jnp.float32),
            pltpu.VMEM((NPAD,), jnp.float32),
            pltpu.VMEM((NT, COLS16), jnp.float32),
            pltpu.VMEM((COLS16,), jnp.float32),
        ],
        compiler_params=_SC_PARAMS,
    )
    tcb_call = pl.pallas_call(
        _tcb_body,
        out_shape=jax.ShapeDtypeStruct((2, NPAD), jnp.float32),
    )
    return deg_call, tca_call, msg_call, tcb_call


def kernel(g, x, W_est, b_est, W_gnn, b_gnn, W_cls, b_cls):
    deg_call, tca_call, msg_call, tcb_call = _build_calls()
    _, dinv = deg_call(g)
    xw, biasrow = tca_call(x, W_est, W_gnn, W_cls,
                           b_est.reshape(1, 1), b_gnn.reshape(1, -1),
                           b_cls.reshape(1, 1))
    _, psum = msg_call(g, xw, dinv)
    out2 = tcb_call(psum, biasrow)
    s = out2[0, :N].reshape(N, 1)
    y = out2[1, :N].reshape(N, 1)
    return (y, s)
